# Initial kernel scaffold; baseline (speedup 1.0000x reference)
import jax
import jax.numpy as jnp
from jax.experimental import pallas as pl


def kernel(x, edge_index, edge_attr, W1, b1, W2, b2, W3, b3):
    raise NotImplementedError("write your pallas kernel here")


# speedup vs baseline: 4.6934x; 4.6934x over previous
import functools

import jax
import jax.numpy as jnp
from jax import lax
from jax.experimental import pallas as pl
from jax.experimental.pallas import tpu as pltpu
from jax.experimental.pallas import tpu_sc as plsc

_NC = 2
_NS = 16
_SUB = 128
_GRP = 8


def _scatter_partials(edge_attr, col2d, n_pad):
    n_edges, edge_dim = edge_attr.shape
    total_grp = n_edges // (_SUB * _GRP)
    nw = _NC * _NS
    rows_per_tile = n_pad // _NS

    zeros = jnp.zeros((rows_per_tile, edge_dim), jnp.float32)
    mesh = plsc.VectorSubcoreMesh(core_axis_name="c", subcore_axis_name="s")

    @functools.partial(
        pl.kernel,
        out_type=jax.ShapeDtypeStruct((_NC * n_pad, edge_dim), jnp.float32),
        mesh=mesh,
        scratch_types=[
            pltpu.VMEM((_GRP, _SUB), jnp.int32),
            pltpu.VMEM((_SUB, edge_dim), jnp.float32),
            pltpu.VMEM_SHARED((n_pad, edge_dim), jnp.float32),
        ],
        compiler_params=pltpu.CompilerParams(use_tc_tiling_on_sc=False),
    )
    def scatter_kernel(attr_hbm, col_hbm, zero_hbm, out_hbm, idx_v, attr_v,
                       agg_sh):
        c = lax.axis_index("c")
        s = lax.axis_index("s")
        t = c * _NS + s
        r0 = s * rows_per_tile
        pltpu.sync_copy(zero_hbm, agg_sh.at[pl.ds(r0, rows_per_tile)])
        plsc.subcore_barrier()

        start_grp = t * total_grp // nw
        n_grp = (t + 1) * total_grp // nw - start_grp

        @pl.loop(0, n_grp)
        def _(i):
            g = (start_grp + i) * _GRP
            pltpu.sync_copy(col_hbm.at[pl.ds(g, _GRP)], idx_v)
            for j in range(_GRP):
                pltpu.sync_copy(attr_hbm.at[pl.ds((g + j) * _SUB, _SUB)], attr_v)
                pltpu.sync_copy(attr_v, agg_sh.at[idx_v.at[j]], add=True)

        plsc.subcore_barrier()
        pltpu.sync_copy(agg_sh.at[pl.ds(r0, rows_per_tile)],
                        out_hbm.at[pl.ds(c * n_pad + r0, rows_per_tile)])

    return scatter_kernel(edge_attr, col2d, zeros).reshape(_NC, n_pad, edge_dim)


def _mlp(x, agg2, W1x, W1a, b1, W2, b2, W3, b3, blk):
    n_nodes, node_dim = x.shape
    edge_dim = agg2.shape[-1]
    hidden = W2.shape[0]
    out_dim = W3.shape[1]

    def body(x_ref, a_ref, w1x_ref, w1a_ref, b1_ref, w2_ref, b2_ref,
             w3_ref, b3_ref, out_ref):
        agg = a_ref[0] + a_ref[1]
        h = jnp.dot(x_ref[...], w1x_ref[...], preferred_element_type=jnp.float32)
        h = h + jnp.dot(agg, w1a_ref[...], preferred_element_type=jnp.float32)
        h = jnp.maximum(h + b1_ref[...], 0.0)
        h = jnp.dot(h, w2_ref[...], preferred_element_type=jnp.float32)
        h = jnp.maximum(h + b2_ref[...], 0.0)
        out_ref[...] = (jnp.dot(h, w3_ref[...], preferred_element_type=jnp.float32)
                        + b3_ref[...])

    return pl.pallas_call(
        body,
        grid=(n_nodes // blk,),
        in_specs=[
            pl.BlockSpec((blk, node_dim), lambda i: (i, 0)),
            pl.BlockSpec((_NC, blk, edge_dim), lambda i: (0, i, 0)),
            pl.BlockSpec((node_dim, hidden), lambda i: (0, 0)),
            pl.BlockSpec((edge_dim, hidden), lambda i: (0, 0)),
            pl.BlockSpec((1, hidden), lambda i: (0, 0)),
            pl.BlockSpec((hidden, hidden), lambda i: (0, 0)),
            pl.BlockSpec((1, hidden), lambda i: (0, 0)),
            pl.BlockSpec((hidden, out_dim), lambda i: (0, 0)),
            pl.BlockSpec((1, out_dim), lambda i: (0, 0)),
        ],
        out_specs=pl.BlockSpec((blk, out_dim), lambda i: (i, 0)),
        out_shape=jax.ShapeDtypeStruct((n_nodes, out_dim), jnp.float32),
    )(x, agg2, W1x, W1a, b1.reshape(1, -1), W2, b2.reshape(1, -1),
      W3, b3.reshape(1, -1))


def kernel(x, edge_index, edge_attr, W1, b1, W2, b2, W3, b3):
    n_nodes, node_dim = x.shape
    n_edges = edge_attr.shape[0]
    assert n_edges % (_SUB * _GRP) == 0
    n_pad = -(-n_nodes // (_NS * 8)) * (_NS * 8)
    col2d = edge_index[1].astype(jnp.int32).reshape(n_edges // _SUB, _SUB)
    agg2 = _scatter_partials(edge_attr, col2d, n_pad)
    return _mlp(x, agg2, W1[:node_dim], W1[node_dim:], b1, W2, b2, W3, b3,
                blk=2000)


# speedup vs baseline: 5.9413x; 1.2659x over previous
import functools

import jax
import jax.numpy as jnp
from jax import lax
from jax.experimental import pallas as pl
from jax.experimental.pallas import tpu as pltpu
from jax.experimental.pallas import tpu_sc as plsc

_NC = 2
_NS = 16
_SUB = 128
_GRP = 8


_CHUNK = _SUB * _GRP


def _scatter_partials(edge_attr, col, n_pad):
    n_edges, edge_dim = edge_attr.shape
    total_grp = n_edges // _CHUNK
    nw = _NC * _NS
    rows_per_tile = n_pad // _NS

    zeros = jnp.zeros((rows_per_tile, edge_dim), jnp.float32)
    mesh = plsc.VectorSubcoreMesh(core_axis_name="c", subcore_axis_name="s")

    @functools.partial(
        pl.kernel,
        out_type=jax.ShapeDtypeStruct((_NC * n_pad, edge_dim), jnp.float32),
        mesh=mesh,
        scratch_types=[
            pltpu.VMEM((_CHUNK,), jnp.int32),
            pltpu.VMEM((_CHUNK, edge_dim), jnp.float32),
            pltpu.VMEM_SHARED((n_pad, edge_dim), jnp.float32),
        ],
        compiler_params=pltpu.CompilerParams(use_tc_tiling_on_sc=False),
    )
    def scatter_kernel(attr_hbm, col_hbm, zero_hbm, out_hbm, idx_v, attr_v,
                       agg_sh):
        c = lax.axis_index("c")
        s = lax.axis_index("s")
        t = c * _NS + s
        r0 = s * rows_per_tile
        pltpu.sync_copy(zero_hbm, agg_sh.at[pl.ds(r0, rows_per_tile)])
        plsc.subcore_barrier()

        start_grp = t * total_grp // nw
        n_grp = (t + 1) * total_grp // nw - start_grp

        @pl.loop(0, n_grp)
        def _(i):
            g = (start_grp + i) * _CHUNK
            pltpu.sync_copy(col_hbm.at[pl.ds(g, _CHUNK)], idx_v)
            pltpu.sync_copy(attr_hbm.at[pl.ds(g, _CHUNK)], attr_v)
            pltpu.sync_copy(attr_v, agg_sh.at[idx_v], add=True)

        plsc.subcore_barrier()
        pltpu.sync_copy(agg_sh.at[pl.ds(r0, rows_per_tile)],
                        out_hbm.at[pl.ds(c * n_pad + r0, rows_per_tile)])

    return scatter_kernel(edge_attr, col, zeros)


def _mlp(x, agg2, W1x, W1a, b1, W2, b2, W3, b3, blk):
    n_nodes, node_dim = x.shape
    edge_dim = agg2.shape[-1]
    hidden = W2.shape[0]
    out_dim = W3.shape[1]

    def body(x_ref, a_ref, w1x_ref, w1a_ref, b1_ref, w2_ref, b2_ref,
             w3_ref, b3_ref, out_ref):
        agg = a_ref[0] + a_ref[1]
        h = jnp.dot(x_ref[...], w1x_ref[...], preferred_element_type=jnp.float32)
        h = h + jnp.dot(agg, w1a_ref[...], preferred_element_type=jnp.float32)
        h = jnp.maximum(h + b1_ref[...], 0.0)
        h = jnp.dot(h, w2_ref[...], preferred_element_type=jnp.float32)
        h = jnp.maximum(h + b2_ref[...], 0.0)
        out_ref[...] = (jnp.dot(h, w3_ref[...], preferred_element_type=jnp.float32)
                        + b3_ref[...])

    return pl.pallas_call(
        body,
        grid=(n_nodes // blk,),
        in_specs=[
            pl.BlockSpec((blk, node_dim), lambda i: (i, 0)),
            pl.BlockSpec((_NC, blk, edge_dim), lambda i: (0, i, 0)),
            pl.BlockSpec((node_dim, hidden), lambda i: (0, 0)),
            pl.BlockSpec((edge_dim, hidden), lambda i: (0, 0)),
            pl.BlockSpec((1, hidden), lambda i: (0, 0)),
            pl.BlockSpec((hidden, hidden), lambda i: (0, 0)),
            pl.BlockSpec((1, hidden), lambda i: (0, 0)),
            pl.BlockSpec((hidden, out_dim), lambda i: (0, 0)),
            pl.BlockSpec((1, out_dim), lambda i: (0, 0)),
        ],
        out_specs=pl.BlockSpec((blk, out_dim), lambda i: (i, 0)),
        out_shape=jax.ShapeDtypeStruct((n_nodes, out_dim), jnp.float32),
    )(x, agg2, W1x, W1a, b1.reshape(1, -1), W2, b2.reshape(1, -1),
      W3, b3.reshape(1, -1))


def kernel(x, edge_index, edge_attr, W1, b1, W2, b2, W3, b3):
    n_nodes, node_dim = x.shape
    n_edges = edge_attr.shape[0]
    assert n_edges % _CHUNK == 0
    n_pad = -(-n_nodes // (_NS * 8)) * (_NS * 8)
    col = edge_index[1].astype(jnp.int32)
    agg2 = _scatter_partials(edge_attr, col, n_pad).reshape(_NC, n_pad, -1)
    return _mlp(x, agg2, W1[:node_dim], W1[node_dim:], b1, W2, b2, W3, b3,
                blk=2000)


# speedup vs baseline: 13.8182x; 2.3258x over previous
import functools

import jax
import jax.numpy as jnp
from jax import lax
from jax.experimental import pallas as pl
from jax.experimental.pallas import tpu as pltpu
from jax.experimental.pallas import tpu_sc as plsc

_NC = 2
_NS = 16
_SUB = 128
_GRP = 8


_CH = 400


def _scatter_partials(attr_t, col, n_pad):
    edge_dim, n_edges = attr_t.shape
    nw = _NC * _NS
    edges_per_tile = n_edges // nw
    n_ch = edges_per_tile // _CH
    assert n_ch * _CH == edges_per_tile and n_ch % 2 == 0 and _CH % 16 == 0
    rows_per_tile = n_pad // _NS

    zeros = jnp.zeros((rows_per_tile, edge_dim), jnp.float32)
    mesh = plsc.VectorSubcoreMesh(core_axis_name="c", subcore_axis_name="s")

    @functools.partial(
        pl.kernel,
        out_type=jax.ShapeDtypeStruct((_NC * n_pad, edge_dim), jnp.float32),
        mesh=mesh,
        scratch_types=[
            pltpu.VMEM((2, edge_dim, _CH), jnp.float32),
            pltpu.VMEM((2, _CH, edge_dim), jnp.float32),
            pltpu.VMEM((4, _CH), jnp.int32),
            pltpu.VMEM_SHARED((n_pad, edge_dim), jnp.float32),
            pltpu.SemaphoreType.DMA,
            pltpu.SemaphoreType.DMA,
            pltpu.SemaphoreType.DMA,
            pltpu.SemaphoreType.DMA,
        ],
        compiler_params=pltpu.CompilerParams(use_tc_tiling_on_sc=False,
                                             needs_layout_passes=False),
    )
    def scatter_kernel(attr_hbm, col_hbm, zero_hbm, out_hbm,
                       at_v, ar_v, idx_v, agg_sh, l0, l1, s0, s1):
        lsem = (l0, l1)
        ssem = (s0, s1)
        c = lax.axis_index("c")
        s = lax.axis_index("s")
        t = c * _NS + s
        r0 = s * rows_per_tile
        pltpu.sync_copy(zero_hbm, agg_sh.at[pl.ds(r0, rows_per_tile)])
        plsc.subcore_barrier()

        e_base = t * edges_per_tile

        def load(ch, b):
            e0 = e_base + ch * _CH
            pltpu.async_copy(attr_hbm.at[:, pl.ds(e0, _CH)], at_v.at[b], lsem[b])
            pltpu.async_copy(col_hbm.at[pl.ds(e0, _CH)], idx_v.at[ch % 4], lsem[b])

        def wait_load(ch, b):
            e0 = e_base + ch * _CH
            pltpu.make_async_copy(attr_hbm.at[:, pl.ds(e0, _CH)], at_v.at[b],
                                  lsem[b]).wait()
            pltpu.make_async_copy(col_hbm.at[pl.ds(e0, _CH)], idx_v.at[ch % 4],
                                  lsem[b]).wait()

        def drain_scatter(b):
            pltpu.make_async_copy(ar_v.at[b], agg_sh.at[pl.ds(0, _CH)],
                                  ssem[b]).wait()

        def transpose(b):
            @pl.loop(0, _CH // 16)
            def _(e16):
                rows = e16 * 16 + lax.iota(jnp.int32, 16)
                for f in range(edge_dim):
                    v = at_v.at[b].at[f][pl.ds(e16 * 16, 16)]
                    cols = jnp.full((16,), f, jnp.int32)
                    plsc.store_scatter(ar_v.at[b], [rows, cols], v)

        load(0, 0)
        load(1, 1)

        @pl.loop(0, n_ch // 2)
        def _(p):
            for k in range(2):
                ch = 2 * p + k
                b = k
                wait_load(ch, b)

                @pl.when(ch >= 2)
                def _():
                    drain_scatter(b)

                transpose(b)
                pltpu.async_copy(ar_v.at[b], agg_sh.at[idx_v.at[ch % 4]],
                                 ssem[b], add=True)

                @pl.when(ch + 2 < n_ch)
                def _():
                    load(ch + 2, b)

        drain_scatter(0)
        drain_scatter(1)

        plsc.subcore_barrier()
        pltpu.sync_copy(agg_sh.at[pl.ds(r0, rows_per_tile)],
                        out_hbm.at[pl.ds(c * n_pad + r0, rows_per_tile)])

    return scatter_kernel(attr_t, col, zeros)


def _mlp(x, agg2, W1x, W1a, b1, W2, b2, W3, b3, blk):
    n_nodes, node_dim = x.shape
    edge_dim = agg2.shape[-1]
    hidden = W2.shape[0]
    out_dim = W3.shape[1]

    def body(x_ref, a_ref, w1x_ref, w1a_ref, b1_ref, w2_ref, b2_ref,
             w3_ref, b3_ref, out_ref):
        agg = a_ref[0] + a_ref[1]
        h = jnp.dot(x_ref[...], w1x_ref[...], preferred_element_type=jnp.float32)
        h = h + jnp.dot(agg, w1a_ref[...], preferred_element_type=jnp.float32)
        h = jnp.maximum(h + b1_ref[...], 0.0)
        h = jnp.dot(h, w2_ref[...], preferred_element_type=jnp.float32)
        h = jnp.maximum(h + b2_ref[...], 0.0)
        out_ref[...] = (jnp.dot(h, w3_ref[...], preferred_element_type=jnp.float32)
                        + b3_ref[...])

    return pl.pallas_call(
        body,
        grid=(n_nodes // blk,),
        in_specs=[
            pl.BlockSpec((blk, node_dim), lambda i: (i, 0)),
            pl.BlockSpec((_NC, blk, edge_dim), lambda i: (0, i, 0)),
            pl.BlockSpec((node_dim, hidden), lambda i: (0, 0)),
            pl.BlockSpec((edge_dim, hidden), lambda i: (0, 0)),
            pl.BlockSpec((1, hidden), lambda i: (0, 0)),
            pl.BlockSpec((hidden, hidden), lambda i: (0, 0)),
            pl.BlockSpec((1, hidden), lambda i: (0, 0)),
            pl.BlockSpec((hidden, out_dim), lambda i: (0, 0)),
            pl.BlockSpec((1, out_dim), lambda i: (0, 0)),
        ],
        out_specs=pl.BlockSpec((blk, out_dim), lambda i: (i, 0)),
        out_shape=jax.ShapeDtypeStruct((n_nodes, out_dim), jnp.float32),
    )(x, agg2, W1x, W1a, b1.reshape(1, -1), W2, b2.reshape(1, -1),
      W3, b3.reshape(1, -1))


def kernel(x, edge_index, edge_attr, W1, b1, W2, b2, W3, b3):
    n_nodes, node_dim = x.shape
    n_pad = -(-n_nodes // (_NS * 8)) * (_NS * 8)
    col = edge_index[1].astype(jnp.int32)
    agg2 = _scatter_partials(edge_attr.T, col, n_pad).reshape(_NC, n_pad, -1)
    return _mlp(x, agg2, W1[:node_dim], W1[node_dim:], b1, W2, b2, W3, b3,
                blk=2000)


# speedup vs baseline: 17.2255x; 1.2466x over previous
import functools

import jax
import jax.numpy as jnp
from jax import lax
from jax.experimental import pallas as pl
from jax.experimental.pallas import tpu as pltpu
from jax.experimental.pallas import tpu_sc as plsc

_NC = 2
_NS = 16
_SUB = 128
_GRP = 8


_CH = 256


def _scatter_partials(attr4, col, n_pad):
    nb_f, nb_e, _, _ = attr4.shape
    edge_dim = nb_f * 8
    nw = _NC * _NS
    rows_per_tile = n_pad // _NS
    total_pairs = nb_e * 128 // (2 * _CH)

    zeros = jnp.zeros((rows_per_tile, edge_dim), jnp.float32)
    mesh = plsc.VectorSubcoreMesh(core_axis_name="c", subcore_axis_name="s")

    @functools.partial(
        pl.kernel,
        out_type=jax.ShapeDtypeStruct((_NC * n_pad, edge_dim), jnp.float32),
        mesh=mesh,
        scratch_types=[
            pltpu.VMEM((2, nb_f, _CH // 128, 8, 128), jnp.float32),
            pltpu.VMEM((2, _CH, edge_dim), jnp.float32),
            pltpu.VMEM((4, _CH), jnp.int32),
            pltpu.VMEM_SHARED((n_pad, edge_dim), jnp.float32),
            pltpu.SemaphoreType.DMA,
            pltpu.SemaphoreType.DMA,
            pltpu.SemaphoreType.DMA,
            pltpu.SemaphoreType.DMA,
        ],
        compiler_params=pltpu.CompilerParams(use_tc_tiling_on_sc=False,
                                             needs_layout_passes=False),
    )
    def scatter_kernel(attr_hbm, col_hbm, zero_hbm, out_hbm,
                       at_v, ar_v, idx_v, agg_sh, l0, l1, s0, s1):
        lsem = (l0, l1)
        ssem = (s0, s1)
        c = lax.axis_index("c")
        s = lax.axis_index("s")
        t = c * _NS + s
        r0 = s * rows_per_tile
        pltpu.sync_copy(zero_hbm, agg_sh.at[pl.ds(r0, rows_per_tile)])
        plsc.subcore_barrier()

        start_ch = 2 * (t * total_pairs // nw)
        n_ch = 2 * ((t + 1) * total_pairs // nw) - start_ch

        def load(ch, b):
            eb0 = (start_ch + ch) * (_CH // 128)
            for fb in range(nb_f):
                pltpu.async_copy(attr_hbm.at[fb, pl.ds(eb0, _CH // 128)],
                                 at_v.at[b, fb], lsem[b])
            pltpu.async_copy(col_hbm.at[pl.ds(eb0 * 128, _CH)],
                             idx_v.at[ch % 4], lsem[b])

        def wait_load(ch, b):
            eb0 = (start_ch + ch) * (_CH // 128)
            for fb in range(nb_f):
                pltpu.make_async_copy(attr_hbm.at[fb, pl.ds(eb0, _CH // 128)],
                                      at_v.at[b, fb], lsem[b]).wait()
            pltpu.make_async_copy(col_hbm.at[pl.ds(eb0 * 128, _CH)],
                                  idx_v.at[ch % 4], lsem[b]).wait()

        def drain_scatter(b):
            pltpu.make_async_copy(ar_v.at[b], agg_sh.at[pl.ds(0, _CH)],
                                  ssem[b]).wait()

        def transpose(b):
            @pl.loop(0, 8)
            def _(e16):
                base = e16 * 16
                iota = lax.iota(jnp.int32, 16)
                for jb in range(_CH // 128):
                    rows = jb * 128 + base + iota
                    for fb in range(nb_f):
                        for f in range(8):
                            v = at_v.at[b, fb, jb, f][pl.ds(base, 16)]
                            cols = jnp.full((16,), fb * 8 + f, jnp.int32)
                            plsc.store_scatter(ar_v.at[b], [rows, cols], v)

        load(0, 0)
        load(1, 1)

        @pl.loop(0, n_ch // 2)
        def _(p):
            for k in range(2):
                ch = 2 * p + k
                b = k
                wait_load(ch, b)

                @pl.when(ch >= 2)
                def _():
                    drain_scatter(b)

                transpose(b)
                pltpu.async_copy(ar_v.at[b], agg_sh.at[idx_v.at[ch % 4]],
                                 ssem[b], add=True)

                @pl.when(ch + 2 < n_ch)
                def _():
                    load(ch + 2, b)

        drain_scatter(0)
        drain_scatter(1)

        plsc.subcore_barrier()
        pltpu.sync_copy(agg_sh.at[pl.ds(r0, rows_per_tile)],
                        out_hbm.at[pl.ds(c * n_pad + r0, rows_per_tile)])

    return scatter_kernel(attr4, col, zeros)


def _mlp(x, agg2, W1x, W1a, b1, W2, b2, W3, b3, blk):
    n_nodes, node_dim = x.shape
    edge_dim = agg2.shape[-1]
    hidden = W2.shape[0]
    out_dim = W3.shape[1]

    def body(x_ref, a_ref, w1x_ref, w1a_ref, b1_ref, w2_ref, b2_ref,
             w3_ref, b3_ref, out_ref):
        agg = a_ref[0] + a_ref[1]
        h = jnp.dot(x_ref[...], w1x_ref[...], preferred_element_type=jnp.float32)
        h = h + jnp.dot(agg, w1a_ref[...], preferred_element_type=jnp.float32)
        h = jnp.maximum(h + b1_ref[...], 0.0)
        h = jnp.dot(h, w2_ref[...], preferred_element_type=jnp.float32)
        h = jnp.maximum(h + b2_ref[...], 0.0)
        out_ref[...] = (jnp.dot(h, w3_ref[...], preferred_element_type=jnp.float32)
                        + b3_ref[...])

    return pl.pallas_call(
        body,
        grid=(n_nodes // blk,),
        in_specs=[
            pl.BlockSpec((blk, node_dim), lambda i: (i, 0)),
            pl.BlockSpec((_NC, blk, edge_dim), lambda i: (0, i, 0)),
            pl.BlockSpec((node_dim, hidden), lambda i: (0, 0)),
            pl.BlockSpec((edge_dim, hidden), lambda i: (0, 0)),
            pl.BlockSpec((1, hidden), lambda i: (0, 0)),
            pl.BlockSpec((hidden, hidden), lambda i: (0, 0)),
            pl.BlockSpec((1, hidden), lambda i: (0, 0)),
            pl.BlockSpec((hidden, out_dim), lambda i: (0, 0)),
            pl.BlockSpec((1, out_dim), lambda i: (0, 0)),
        ],
        out_specs=pl.BlockSpec((blk, out_dim), lambda i: (i, 0)),
        out_shape=jax.ShapeDtypeStruct((n_nodes, out_dim), jnp.float32),
    )(x, agg2, W1x, W1a, b1.reshape(1, -1), W2, b2.reshape(1, -1),
      W3, b3.reshape(1, -1))


def kernel(x, edge_index, edge_attr, W1, b1, W2, b2, W3, b3):
    n_nodes, node_dim = x.shape
    n_pad = -(-n_nodes // (_NS * 8)) * (_NS * 8)
    col = edge_index[1].astype(jnp.int32)
    n_edges, edge_dim = edge_attr.shape
    attr4 = edge_attr.reshape(n_edges // 128, 128, edge_dim // 8, 8)
    attr4 = attr4.transpose(2, 0, 3, 1)
    agg2 = _scatter_partials(attr4, col, n_pad).reshape(_NC, n_pad, -1)
    return _mlp(x, agg2, W1[:node_dim], W1[node_dim:], b1, W2, b2, W3, b3,
                blk=2000)
